# baseline (speedup 1.0000x reference)
import functools

import jax
import jax.numpy as jnp
from jax import lax
from jax.experimental import pallas as pl
from jax.experimental.pallas import tpu as pltpu
from jax.experimental.pallas import tpu_sc as plsc

_N = 10000
_E = 160000
_EH = _E // 2
_H = 256
_G = 16
_F = 256
_FH = 128
_NS = 16
_K = 40
_EPT = _EH // _NS
_NCH = _EPT // _K
_ZCH = _N // _K


def _lin1_body(x_ref, w1_ref, b1_ref, out_ref):
    r = lax.dot_general(x_ref[...], w1_ref[...], (((1,), (1,)), ((), ())),
                        preferred_element_type=jnp.float32)
    r = r + b1_ref[...]
    out_ref[0] = r[:, :_FH]
    out_ref[1] = r[:, _FH:]


def _lin1(x, W1, b1r):
    nb = 10
    return pl.pallas_call(
        _lin1_body,
        grid=(nb,),
        in_specs=[
            pl.BlockSpec((_N // nb, _H), lambda i: (i, 0)),
            pl.BlockSpec((_F, _H), lambda i: (0, 0)),
            pl.BlockSpec((1, _F), lambda i: (0, 0)),
        ],
        out_specs=pl.BlockSpec((2, _N // nb, _FH), lambda i: (0, i, 0)),
        out_shape=jax.ShapeDtypeStruct((2, _N, _FH), jnp.float32),
    )(x, W1, b1r)


def _mlp_body(ea_ref, wm1_ref, bm1_ref, wm2_ref, bm2_ref, out_ref):
    h = lax.dot_general(ea_ref[...], wm1_ref[...], (((1,), (1,)), ((), ())),
                        preferred_element_type=jnp.float32)
    h = jnp.maximum(h + bm1_ref[...], 0.0)
    ew = lax.dot_general(h, wm2_ref[...], (((1,), (1,)), ((), ())),
                         preferred_element_type=jnp.float32)
    ew = jnp.maximum(ew + bm2_ref[...], 0.0)
    out_ref[0] = ew[:, :_FH]
    out_ref[1] = ew[:, _FH:]


def _mlp(edge_attr, Wm1, bm1r, Wm2, bm2r):
    eb = 40
    return pl.pallas_call(
        _mlp_body,
        grid=(eb,),
        in_specs=[
            pl.BlockSpec((_EH // eb, _G), lambda i: (i, 0)),
            pl.BlockSpec((_F, _G), lambda i: (0, 0)),
            pl.BlockSpec((1, _F), lambda i: (0, 0)),
            pl.BlockSpec((_F, _F), lambda i: (0, 0)),
            pl.BlockSpec((1, _F), lambda i: (0, 0)),
        ],
        out_specs=pl.BlockSpec((2, _EH // eb, _FH), lambda i: (0, i, 0)),
        out_shape=jax.ShapeDtypeStruct((2, _EH, _FH), jnp.float32),
    )(edge_attr, Wm1, bm1r, Wm2, bm2r)


def _sc_body(xi_hbm, ew_hbm, col_hbm, row_hbm, out_hbm,
             colr, rowr, xiv, ewv, accum,
             isem0, isem1, isem2, isem3, gsem0, gsem1, esem0, esem1,
             ssem0, ssem1):
    c = lax.axis_index("c")
    s = lax.axis_index("s")
    isems = (isem0, isem1, isem2, isem3)
    gsems = (gsem0, gsem1)
    esems = (esem0, esem1)
    ssems = (ssem0, ssem1)

    ebase = c * _EH + s * _EPT

    def idx_start(j, q):
        pltpu.async_copy(col_hbm.at[c, s, j], colr.at[q], isems[q])
        pltpu.async_copy(row_hbm.at[s, j], rowr.at[q], isems[q])

    def idx_wait(j, q):
        pltpu.make_async_copy(col_hbm.at[c, s, j], colr.at[q],
                              isems[q]).wait()
        pltpu.make_async_copy(row_hbm.at[s, j], rowr.at[q],
                              isems[q]).wait()

    def data_start(j, b, q):
        pltpu.async_copy(xi_hbm.at[colr.at[q]], xiv.at[b], gsems[b])
        pltpu.async_copy(ew_hbm.at[pl.ds(ebase + j * _K, _K)], ewv.at[b],
                         esems[b])

    def scat_wait(b, q):
        pltpu.make_async_copy(xiv.at[b], accum.at[rowr.at[q]],
                              ssems[b]).wait()

    def data_finish(j, b, q):
        pltpu.make_async_copy(xi_hbm.at[colr.at[q]], xiv.at[b],
                              gsems[b]).wait()
        pltpu.make_async_copy(ew_hbm.at[pl.ds(ebase + j * _K, _K)],
                              ewv.at[b], esems[b]).wait()
        xb = xiv.at[b]
        eb = ewv.at[b]

        @plsc.parallel_loop(0, _K, unroll=4)
        def _(r):
            for g in range(8):
                sl = pl.ds(g * 16, 16)
                xb[r, sl] = xb[r, sl] * eb[r, sl]

        pltpu.async_copy(xb, accum.at[rowr.at[q]], ssems[b], add=True)

    idx_start(0, 0)
    idx_start(1, 1)
    idx_start(2, 2)

    @plsc.parallel_loop(0, _K, unroll=4)
    def _(r):
        for g in range(8):
            ewv[0, r, pl.ds(g * 16, 16)] = jnp.zeros((16,), jnp.float32)

    def zchunk(j, carry):
        z = s + _NS * j

        @pl.when(z < _ZCH)
        def _():
            pltpu.async_copy(ewv.at[0], accum.at[pl.ds(z * _K, _K)], gsem0)

        return carry

    nz = (_ZCH + _NS - 1) // _NS
    lax.fori_loop(0, nz, zchunk, 0)

    def zdrain(j, carry):
        z = s + _NS * j

        @pl.when(z < _ZCH)
        def _():
            pltpu.make_async_copy(ewv.at[0], accum.at[pl.ds(z * _K, _K)],
                                  gsem0).wait()

        return carry

    lax.fori_loop(0, nz, zdrain, 0)
    plsc.subcore_barrier()

    idx_wait(0, 0)
    data_start(0, 0, 0)

    def step(t, carry):
        j0 = 4 * t
        for u in range(4):
            j = j0 + u

            @pl.when(j >= 1)
            def _():
                scat_wait((u + 1) % 2, (u + 3) % 4)

            idx_wait(j + 1, (u + 1) % 4)
            data_start(j + 1, (u + 1) % 2, (u + 1) % 4)
            data_finish(j, u % 2, u)

            @pl.when(j + 3 < _NCH)
            def _():
                idx_start(j + 3, (u + 3) % 4)

        return carry

    lax.fori_loop(0, _NCH // 4, step, 0)
    j = _NCH - 1
    scat_wait(1, 3)
    data_finish(j, 0, 0)
    scat_wait(0, 0)
    plsc.subcore_barrier()

    r0 = s * 624

    @pl.when(s < _NS - 1)
    def _():
        pltpu.sync_copy(accum.at[pl.ds(r0, 624)],
                        out_hbm.at[pl.ds(c * _N + r0, 624)])

    @pl.when(s == _NS - 1)
    def _():
        pltpu.sync_copy(accum.at[pl.ds(15 * 624, _N - 15 * 624)],
                        out_hbm.at[pl.ds(c * _N + 15 * 624, _N - 15 * 624)])


_sc_scatter = functools.partial(
    pl.kernel,
    mesh=plsc.VectorSubcoreMesh(core_axis_name="c", subcore_axis_name="s"),
    out_type=jax.ShapeDtypeStruct((2 * _N, _FH), jnp.float32),
    scratch_types=[
        pltpu.VMEM((4, _K), jnp.int32),
        pltpu.VMEM((4, _K), jnp.int32),
        pltpu.VMEM((2, _K, _FH), jnp.float32),
        pltpu.VMEM((2, _K, _FH), jnp.float32),
        pltpu.VMEM_SHARED((_N, _FH), jnp.float32),
    ] + [pltpu.SemaphoreType.DMA] * 10,
)(_sc_body)


def _lin2a_body(sa_ref, w2_ref, b2_ref, out_ref):
    scat = jnp.concatenate([sa_ref[0], sa_ref[1]], axis=1)
    r = lax.dot_general(scat, w2_ref[...], (((1,), (1,)), ((), ())),
                        preferred_element_type=jnp.float32)
    out_ref[...] = r + b2_ref[...]


def _lin2a(sa3, W2, b2r):
    nb = 10
    return pl.pallas_call(
        _lin2a_body,
        grid=(nb,),
        in_specs=[
            pl.BlockSpec((2, _N // nb, _FH), lambda i: (0, i, 0)),
            pl.BlockSpec((_H, _F), lambda i: (0, 0)),
            pl.BlockSpec((1, _H), lambda i: (0, 0)),
        ],
        out_specs=pl.BlockSpec((_N // nb, _H), lambda i: (i, 0)),
        out_shape=jax.ShapeDtypeStruct((_N, _H), jnp.float32),
    )(sa3, W2, b2r)


def _lin2b_body(sb_ref, pa_ref, w2_ref, out_ref):
    scat = jnp.concatenate([sb_ref[0], sb_ref[1]], axis=1)
    r = lax.dot_general(scat, w2_ref[...], (((1,), (1,)), ((), ())),
                        preferred_element_type=jnp.float32)
    out_ref[...] = r + pa_ref[...]


def _lin2b(sb3, partA, W2):
    nb = 10
    return pl.pallas_call(
        _lin2b_body,
        grid=(nb,),
        in_specs=[
            pl.BlockSpec((2, _N // nb, _FH), lambda i: (0, i, 0)),
            pl.BlockSpec((_N // nb, _H), lambda i: (i, 0)),
            pl.BlockSpec((_H, _F), lambda i: (0, 0)),
        ],
        out_specs=pl.BlockSpec((_N // nb, _H), lambda i: (i, 0)),
        out_shape=jax.ShapeDtypeStruct((_N, _H), jnp.float32),
    )(sb3, partA, W2)


def kernel(x, edge_index, edge_attr, W1, b1, W2, b2, Wm1, bm1, Wm2, bm2):
    row = edge_index[0]
    col = edge_index[1]
    col2 = jnp.concatenate([col, col + _N])
    colA = jnp.stack([col2[h * _E:h * _E + _EH] for h in range(2)])
    colB = jnp.stack([col2[h * _E + _EH:(h + 1) * _E] for h in range(2)])
    colA4 = colA.reshape(2, _NS, _NCH, _K)
    colB4 = colB.reshape(2, _NS, _NCH, _K)
    rowA3 = row[:_EH].reshape(_NS, _NCH, _K)
    rowB3 = row[_EH:].reshape(_NS, _NCH, _K)

    bm1r = bm1.reshape(1, _F)
    bm2r = bm2.reshape(1, _F)

    xi = _lin1(x, W1, b1.reshape(1, _F))
    xif = xi.reshape(2 * _N, _FH)
    ewA = _mlp(edge_attr[:_EH], Wm1, bm1r, Wm2, bm2r)
    scA = _sc_scatter(xif, ewA.reshape(2 * _EH, _FH), colA4, rowA3)
    ewB = _mlp(edge_attr[_EH:], Wm1, bm1r, Wm2, bm2r)
    scB = _sc_scatter(xif, ewB.reshape(2 * _EH, _FH), colB4, rowB3)
    partA = _lin2a(scA.reshape(2, _N, _FH), W2, b2.reshape(1, _H))
    return _lin2b(scB.reshape(2, _N, _FH), partA, W2)


# speedup vs baseline: 1.0276x; 1.0276x over previous
import functools

import jax
import jax.numpy as jnp
from jax import lax
from jax.experimental import pallas as pl
from jax.experimental.pallas import tpu as pltpu
from jax.experimental.pallas import tpu_sc as plsc

_N = 10000
_E = 160000
_EA = 24320
_EB = _E - _EA
_H = 256
_G = 16
_F = 256
_FH = 128
_NS = 16
_K = 40
_ZCH = _N // _K


def _lin1_body(x_ref, w1_ref, b1_ref, out_ref):
    r = lax.dot_general(x_ref[...], w1_ref[...], (((1,), (1,)), ((), ())),
                        preferred_element_type=jnp.float32)
    r = r + b1_ref[...]
    out_ref[0] = r[:, :_FH]
    out_ref[1] = r[:, _FH:]


def _lin1(x, W1, b1r):
    nb = 10
    return pl.pallas_call(
        _lin1_body,
        grid=(nb,),
        in_specs=[
            pl.BlockSpec((_N // nb, _H), lambda i: (i, 0)),
            pl.BlockSpec((_F, _H), lambda i: (0, 0)),
            pl.BlockSpec((1, _F), lambda i: (0, 0)),
        ],
        out_specs=pl.BlockSpec((2, _N // nb, _FH), lambda i: (0, i, 0)),
        out_shape=jax.ShapeDtypeStruct((2, _N, _FH), jnp.float32),
    )(x, W1, b1r)


def _mlp_body(ea_ref, wm1_ref, bm1_ref, wm2_ref, bm2_ref, out_ref):
    h = lax.dot_general(ea_ref[...], wm1_ref[...], (((1,), (1,)), ((), ())),
                        preferred_element_type=jnp.float32)
    h = jnp.maximum(h + bm1_ref[...], 0.0)
    ew = lax.dot_general(h, wm2_ref[...], (((1,), (1,)), ((), ())),
                         preferred_element_type=jnp.float32)
    ew = jnp.maximum(ew + bm2_ref[...], 0.0)
    out_ref[0] = ew[:, :_FH]
    out_ref[1] = ew[:, _FH:]


def _mlp(edge_attr, Wm1, bm1r, Wm2, bm2r, e, eb):
    return pl.pallas_call(
        _mlp_body,
        grid=(eb,),
        in_specs=[
            pl.BlockSpec((e // eb, _G), lambda i: (i, 0)),
            pl.BlockSpec((_F, _G), lambda i: (0, 0)),
            pl.BlockSpec((1, _F), lambda i: (0, 0)),
            pl.BlockSpec((_F, _F), lambda i: (0, 0)),
            pl.BlockSpec((1, _F), lambda i: (0, 0)),
        ],
        out_specs=pl.BlockSpec((2, e // eb, _FH), lambda i: (0, i, 0)),
        out_shape=jax.ShapeDtypeStruct((2, e, _FH), jnp.float32),
    )(edge_attr, Wm1, bm1r, Wm2, bm2r)


def _sc_body(eh, nch, xi_hbm, ew_hbm, col_hbm, row_hbm, out_hbm,
             colr, rowr, xiv, ewv, accum,
             isem0, isem1, isem2, isem3, gsem0, gsem1, esem0, esem1,
             ssem0, ssem1):
    c = lax.axis_index("c")
    s = lax.axis_index("s")
    isems = (isem0, isem1, isem2, isem3)
    gsems = (gsem0, gsem1)
    esems = (esem0, esem1)
    ssems = (ssem0, ssem1)

    ept = eh // _NS
    ebase = c * eh + s * ept

    def idx_start(j, q):
        pltpu.async_copy(col_hbm.at[c, s, j], colr.at[q], isems[q])
        pltpu.async_copy(row_hbm.at[s, j], rowr.at[q], isems[q])

    def idx_wait(j, q):
        pltpu.make_async_copy(col_hbm.at[c, s, j], colr.at[q],
                              isems[q]).wait()
        pltpu.make_async_copy(row_hbm.at[s, j], rowr.at[q],
                              isems[q]).wait()

    def data_start(j, b, q):
        pltpu.async_copy(xi_hbm.at[colr.at[q]], xiv.at[b], gsems[b])
        pltpu.async_copy(ew_hbm.at[pl.ds(ebase + j * _K, _K)], ewv.at[b],
                         esems[b])

    def scat_wait(b, q):
        pltpu.make_async_copy(xiv.at[b], accum.at[rowr.at[q]],
                              ssems[b]).wait()

    def data_finish(j, b, q):
        pltpu.make_async_copy(xi_hbm.at[colr.at[q]], xiv.at[b],
                              gsems[b]).wait()
        pltpu.make_async_copy(ew_hbm.at[pl.ds(ebase + j * _K, _K)],
                              ewv.at[b], esems[b]).wait()
        xb = xiv.at[b]
        eb = ewv.at[b]

        @plsc.parallel_loop(0, _K, unroll=4)
        def _(r):
            for g in range(8):
                sl = pl.ds(g * 16, 16)
                xb[r, sl] = xb[r, sl] * eb[r, sl]

        pltpu.async_copy(xb, accum.at[rowr.at[q]], ssems[b], add=True)

    idx_start(0, 0)
    idx_start(1, 1)
    idx_start(2, 2)

    @plsc.parallel_loop(0, _K, unroll=4)
    def _(r):
        for g in range(8):
            ewv[0, r, pl.ds(g * 16, 16)] = jnp.zeros((16,), jnp.float32)

    def zchunk(j, carry):
        z = s + _NS * j

        @pl.when(z < _ZCH)
        def _():
            pltpu.async_copy(ewv.at[0], accum.at[pl.ds(z * _K, _K)], gsem0)

        return carry

    nz = (_ZCH + _NS - 1) // _NS
    lax.fori_loop(0, nz, zchunk, 0)

    def zdrain(j, carry):
        z = s + _NS * j

        @pl.when(z < _ZCH)
        def _():
            pltpu.make_async_copy(ewv.at[0], accum.at[pl.ds(z * _K, _K)],
                                  gsem0).wait()

        return carry

    lax.fori_loop(0, nz, zdrain, 0)
    plsc.subcore_barrier()

    idx_wait(0, 0)
    data_start(0, 0, 0)

    def step(t, carry):
        j0 = 4 * t
        for u in range(4):
            j = j0 + u

            @pl.when(j >= 1)
            def _():
                scat_wait((u + 1) % 2, (u + 3) % 4)

            @pl.when(j + 1 < nch)
            def _():
                idx_wait(j + 1, (u + 1) % 4)
                data_start(j + 1, (u + 1) % 2, (u + 1) % 4)

            data_finish(j, u % 2, u)

            @pl.when(j + 3 < nch)
            def _():
                idx_start(j + 3, (u + 3) % 4)

        return carry

    lax.fori_loop(0, nch // 4, step, 0)
    if nch % 4 == 2:
        j = nch - 2
        scat_wait(1, 3)
        idx_wait(j + 1, 1)
        data_start(j + 1, 1, 1)
        data_finish(j, 0, 0)
        scat_wait(0, 0)
        data_finish(j + 1, 1, 1)
        scat_wait(1, 1)
    else:
        scat_wait(1, 3)
    plsc.subcore_barrier()

    r0 = s * 624

    @pl.when(s < _NS - 1)
    def _():
        pltpu.sync_copy(accum.at[pl.ds(r0, 624)],
                        out_hbm.at[pl.ds(c * _N + r0, 624)])

    @pl.when(s == _NS - 1)
    def _():
        pltpu.sync_copy(accum.at[pl.ds(15 * 624, _N - 15 * 624)],
                        out_hbm.at[pl.ds(c * _N + 15 * 624, _N - 15 * 624)])


def _make_sc_scatter(eh):
    nch = eh // _NS // _K
    return functools.partial(
        pl.kernel,
        mesh=plsc.VectorSubcoreMesh(core_axis_name="c",
                                    subcore_axis_name="s"),
        out_type=jax.ShapeDtypeStruct((2 * _N, _FH), jnp.float32),
        scratch_types=[
            pltpu.VMEM((4, _K), jnp.int32),
            pltpu.VMEM((4, _K), jnp.int32),
            pltpu.VMEM((2, _K, _FH), jnp.float32),
            pltpu.VMEM((2, _K, _FH), jnp.float32),
            pltpu.VMEM_SHARED((_N, _FH), jnp.float32),
        ] + [pltpu.SemaphoreType.DMA] * 10,
    )(functools.partial(_sc_body, eh, nch))


_sc_scatter_a = _make_sc_scatter(_EA)
_sc_scatter_b = _make_sc_scatter(_EB)


def _lin2_body(sa_ref, sb_ref, w2_ref, b2_ref, out_ref):
    scat = jnp.concatenate([sa_ref[0] + sb_ref[0], sa_ref[1] + sb_ref[1]],
                           axis=1)
    r = lax.dot_general(scat, w2_ref[...], (((1,), (1,)), ((), ())),
                        preferred_element_type=jnp.float32)
    out_ref[...] = r + b2_ref[...]


def _lin2(sa3, sb3, W2, b2r):
    nb = 10
    return pl.pallas_call(
        _lin2_body,
        grid=(nb,),
        in_specs=[
            pl.BlockSpec((2, _N // nb, _FH), lambda i: (0, i, 0)),
            pl.BlockSpec((2, _N // nb, _FH), lambda i: (0, i, 0)),
            pl.BlockSpec((_H, _F), lambda i: (0, 0)),
            pl.BlockSpec((1, _H), lambda i: (0, 0)),
        ],
        out_specs=pl.BlockSpec((_N // nb, _H), lambda i: (i, 0)),
        out_shape=jax.ShapeDtypeStruct((_N, _H), jnp.float32),
    )(sa3, sb3, W2, b2r)


def kernel(x, edge_index, edge_attr, W1, b1, W2, b2, Wm1, bm1, Wm2, bm2):
    row = edge_index[0]
    col = edge_index[1]
    colA = jnp.stack([col[:_EA], col[:_EA] + _N])
    colB = jnp.stack([col[_EA:], col[_EA:] + _N])
    colA4 = colA.reshape(2, _NS, -1, _K)
    colB4 = colB.reshape(2, _NS, -1, _K)
    rowA3 = row[:_EA].reshape(_NS, -1, _K)
    rowB3 = row[_EA:].reshape(_NS, -1, _K)

    bm1r = bm1.reshape(1, _F)
    bm2r = bm2.reshape(1, _F)

    xi = _lin1(x, W1, b1.reshape(1, _F))
    xif = xi.reshape(2 * _N, _FH)
    ewA = _mlp(edge_attr[:_EA], Wm1, bm1r, Wm2, bm2r, _EA, 16)
    scA = _sc_scatter_a(xif, ewA.reshape(2 * _EA, _FH), colA4, rowA3)
    ewB = _mlp(edge_attr[_EA:], Wm1, bm1r, Wm2, bm2r, _EB, 40)
    scB = _sc_scatter_b(xif, ewB.reshape(2 * _EB, _FH), colB4, rowB3)
    return _lin2(scA.reshape(2, _N, _FH), scB.reshape(2, _N, _FH),
                 W2, b2.reshape(1, _H))


# speedup vs baseline: 1.0577x; 1.0294x over previous
import functools

import jax
import jax.numpy as jnp
from jax import lax
from jax.experimental import pallas as pl
from jax.experimental.pallas import tpu as pltpu
from jax.experimental.pallas import tpu_sc as plsc

_N = 10000
_E = 160000
_EA = 24320
_EB = _E - _EA
_H = 256
_G = 16
_F = 256
_FH = 128
_NS = 16
_K = 40
_ZCH = _N // _K


def _pre_body(x_ref, ea_ref, w1_ref, b1_ref, wm1_ref, bm1_ref, wm2_ref,
              bm2_ref, xi_ref, ew_ref):
    r = lax.dot_general(x_ref[...], w1_ref[...], (((1,), (1,)), ((), ())),
                        preferred_element_type=jnp.float32)
    r = r + b1_ref[...]
    xi_ref[0] = r[:, :_FH]
    xi_ref[1] = r[:, _FH:]
    h = lax.dot_general(ea_ref[...], wm1_ref[...], (((1,), (1,)), ((), ())),
                        preferred_element_type=jnp.float32)
    h = jnp.maximum(h + bm1_ref[...], 0.0)
    ew = lax.dot_general(h, wm2_ref[...], (((1,), (1,)), ((), ())),
                         preferred_element_type=jnp.float32)
    ew = jnp.maximum(ew + bm2_ref[...], 0.0)
    ew_ref[0] = ew[:, :_FH]
    ew_ref[1] = ew[:, _FH:]


def _pre(x, eaA, W1, b1r, Wm1, bm1r, Wm2, bm2r):
    nb = 10
    return pl.pallas_call(
        _pre_body,
        grid=(nb,),
        in_specs=[
            pl.BlockSpec((_N // nb, _H), lambda i: (i, 0)),
            pl.BlockSpec((_EA // nb, _G), lambda i: (i, 0)),
            pl.BlockSpec((_F, _H), lambda i: (0, 0)),
            pl.BlockSpec((1, _F), lambda i: (0, 0)),
            pl.BlockSpec((_F, _G), lambda i: (0, 0)),
            pl.BlockSpec((1, _F), lambda i: (0, 0)),
            pl.BlockSpec((_F, _F), lambda i: (0, 0)),
            pl.BlockSpec((1, _F), lambda i: (0, 0)),
        ],
        out_specs=[
            pl.BlockSpec((2, _N // nb, _FH), lambda i: (0, i, 0)),
            pl.BlockSpec((2, _EA // nb, _FH), lambda i: (0, i, 0)),
        ],
        out_shape=[
            jax.ShapeDtypeStruct((2, _N, _FH), jnp.float32),
            jax.ShapeDtypeStruct((2, _EA, _FH), jnp.float32),
        ],
    )(x, eaA, W1, b1r, Wm1, bm1r, Wm2, bm2r)


def _mlp_body(ea_ref, wm1_ref, bm1_ref, wm2_ref, bm2_ref, out_ref):
    h = lax.dot_general(ea_ref[...], wm1_ref[...], (((1,), (1,)), ((), ())),
                        preferred_element_type=jnp.float32)
    h = jnp.maximum(h + bm1_ref[...], 0.0)
    ew = lax.dot_general(h, wm2_ref[...], (((1,), (1,)), ((), ())),
                         preferred_element_type=jnp.float32)
    ew = jnp.maximum(ew + bm2_ref[...], 0.0)
    out_ref[0] = ew[:, :_FH]
    out_ref[1] = ew[:, _FH:]


def _mlp(edge_attr, Wm1, bm1r, Wm2, bm2r, e, eb):
    return pl.pallas_call(
        _mlp_body,
        grid=(eb,),
        in_specs=[
            pl.BlockSpec((e // eb, _G), lambda i: (i, 0)),
            pl.BlockSpec((_F, _G), lambda i: (0, 0)),
            pl.BlockSpec((1, _F), lambda i: (0, 0)),
            pl.BlockSpec((_F, _F), lambda i: (0, 0)),
            pl.BlockSpec((1, _F), lambda i: (0, 0)),
        ],
        out_specs=pl.BlockSpec((2, e // eb, _FH), lambda i: (0, i, 0)),
        out_shape=jax.ShapeDtypeStruct((2, e, _FH), jnp.float32),
    )(edge_attr, Wm1, bm1r, Wm2, bm2r)


def _sc_body(eh, nch, xi_hbm, ew_hbm, col_hbm, row_hbm, out_hbm,
             colr, rowr, xiv, ewv, accum,
             isem0, isem1, isem2, isem3, gsem0, gsem1, esem0, esem1,
             ssem0, ssem1):
    c = lax.axis_index("c")
    s = lax.axis_index("s")
    isems = (isem0, isem1, isem2, isem3)
    gsems = (gsem0, gsem1)
    esems = (esem0, esem1)
    ssems = (ssem0, ssem1)

    ept = eh // _NS
    ebase = c * eh + s * ept

    def idx_start(j, q):
        pltpu.async_copy(col_hbm.at[c, s, j], colr.at[q], isems[q])
        pltpu.async_copy(row_hbm.at[s, j], rowr.at[q], isems[q])

    def idx_wait(j, q):
        pltpu.make_async_copy(col_hbm.at[c, s, j], colr.at[q],
                              isems[q]).wait()
        pltpu.make_async_copy(row_hbm.at[s, j], rowr.at[q],
                              isems[q]).wait()

    def data_start(j, b, q):
        pltpu.async_copy(xi_hbm.at[colr.at[q]], xiv.at[b], gsems[b])
        pltpu.async_copy(ew_hbm.at[pl.ds(ebase + j * _K, _K)], ewv.at[b],
                         esems[b])

    def scat_wait(b, q):
        pltpu.make_async_copy(xiv.at[b], accum.at[rowr.at[q]],
                              ssems[b]).wait()

    def data_finish(j, b, q):
        pltpu.make_async_copy(xi_hbm.at[colr.at[q]], xiv.at[b],
                              gsems[b]).wait()
        pltpu.make_async_copy(ew_hbm.at[pl.ds(ebase + j * _K, _K)],
                              ewv.at[b], esems[b]).wait()
        xb = xiv.at[b]
        eb = ewv.at[b]

        @plsc.parallel_loop(0, _K, unroll=4)
        def _(r):
            for g in range(8):
                sl = pl.ds(g * 16, 16)
                xb[r, sl] = xb[r, sl] * eb[r, sl]

        pltpu.async_copy(xb, accum.at[rowr.at[q]], ssems[b], add=True)

    idx_start(0, 0)
    idx_start(1, 1)
    idx_start(2, 2)

    @plsc.parallel_loop(0, _K, unroll=4)
    def _(r):
        for g in range(8):
            ewv[0, r, pl.ds(g * 16, 16)] = jnp.zeros((16,), jnp.float32)

    def zchunk(j, carry):
        z = s + _NS * j

        @pl.when(z < _ZCH)
        def _():
            pltpu.async_copy(ewv.at[0], accum.at[pl.ds(z * _K, _K)], gsem0)

        return carry

    nz = (_ZCH + _NS - 1) // _NS
    lax.fori_loop(0, nz, zchunk, 0)

    def zdrain(j, carry):
        z = s + _NS * j

        @pl.when(z < _ZCH)
        def _():
            pltpu.make_async_copy(ewv.at[0], accum.at[pl.ds(z * _K, _K)],
                                  gsem0).wait()

        return carry

    lax.fori_loop(0, nz, zdrain, 0)
    plsc.subcore_barrier()

    idx_wait(0, 0)
    data_start(0, 0, 0)

    def step(t, carry):
        j0 = 4 * t
        for u in range(4):
            j = j0 + u

            @pl.when(j >= 1)
            def _():
                scat_wait((u + 1) % 2, (u + 3) % 4)

            @pl.when(j + 1 < nch)
            def _():
                idx_wait(j + 1, (u + 1) % 4)
                data_start(j + 1, (u + 1) % 2, (u + 1) % 4)

            data_finish(j, u % 2, u)

            @pl.when(j + 3 < nch)
            def _():
                idx_start(j + 3, (u + 3) % 4)

        return carry

    lax.fori_loop(0, nch // 4, step, 0)
    if nch % 4 == 2:
        j = nch - 2
        scat_wait(1, 3)
        idx_wait(j + 1, 1)
        data_start(j + 1, 1, 1)
        data_finish(j, 0, 0)
        scat_wait(0, 0)
        data_finish(j + 1, 1, 1)
        scat_wait(1, 1)
    else:
        scat_wait(1, 3)
    plsc.subcore_barrier()

    r0 = s * 624

    @pl.when(s < _NS - 1)
    def _():
        pltpu.sync_copy(accum.at[pl.ds(r0, 624)],
                        out_hbm.at[pl.ds(c * _N + r0, 624)])

    @pl.when(s == _NS - 1)
    def _():
        pltpu.sync_copy(accum.at[pl.ds(15 * 624, _N - 15 * 624)],
                        out_hbm.at[pl.ds(c * _N + 15 * 624, _N - 15 * 624)])


def _make_sc_scatter(eh):
    nch = eh // _NS // _K
    return functools.partial(
        pl.kernel,
        mesh=plsc.VectorSubcoreMesh(core_axis_name="c",
                                    subcore_axis_name="s"),
        out_type=jax.ShapeDtypeStruct((2 * _N, _FH), jnp.float32),
        scratch_types=[
            pltpu.VMEM((4, _K), jnp.int32),
            pltpu.VMEM((4, _K), jnp.int32),
            pltpu.VMEM((2, _K, _FH), jnp.float32),
            pltpu.VMEM((2, _K, _FH), jnp.float32),
            pltpu.VMEM_SHARED((_N, _FH), jnp.float32),
        ] + [pltpu.SemaphoreType.DMA] * 10,
    )(functools.partial(_sc_body, eh, nch))


_sc_scatter_a = _make_sc_scatter(_EA)
_sc_scatter_b = _make_sc_scatter(_EB)


def _lin2_body(sa_ref, sb_ref, w2_ref, b2_ref, out_ref):
    scat = jnp.concatenate([sa_ref[0] + sb_ref[0], sa_ref[1] + sb_ref[1]],
                           axis=1)
    r = lax.dot_general(scat, w2_ref[...], (((1,), (1,)), ((), ())),
                        preferred_element_type=jnp.float32)
    out_ref[...] = r + b2_ref[...]


def _lin2(sa3, sb3, W2, b2r):
    nb = 10
    return pl.pallas_call(
        _lin2_body,
        grid=(nb,),
        in_specs=[
            pl.BlockSpec((2, _N // nb, _FH), lambda i: (0, i, 0)),
            pl.BlockSpec((2, _N // nb, _FH), lambda i: (0, i, 0)),
            pl.BlockSpec((_H, _F), lambda i: (0, 0)),
            pl.BlockSpec((1, _H), lambda i: (0, 0)),
        ],
        out_specs=pl.BlockSpec((_N // nb, _H), lambda i: (i, 0)),
        out_shape=jax.ShapeDtypeStruct((_N, _H), jnp.float32),
    )(sa3, sb3, W2, b2r)


def kernel(x, edge_index, edge_attr, W1, b1, W2, b2, Wm1, bm1, Wm2, bm2):
    row = edge_index[0]
    col = edge_index[1]
    colA = jnp.stack([col[:_EA], col[:_EA] + _N])
    colB = jnp.stack([col[_EA:], col[_EA:] + _N])
    colA4 = colA.reshape(2, _NS, -1, _K)
    colB4 = colB.reshape(2, _NS, -1, _K)
    rowA3 = row[:_EA].reshape(_NS, -1, _K)
    rowB3 = row[_EA:].reshape(_NS, -1, _K)

    bm1r = bm1.reshape(1, _F)
    bm2r = bm2.reshape(1, _F)

    xi, ewA = _pre(x, edge_attr[:_EA], W1, b1.reshape(1, _F),
                   Wm1, bm1r, Wm2, bm2r)
    xif = xi.reshape(2 * _N, _FH)
    scA = _sc_scatter_a(xif, ewA.reshape(2 * _EA, _FH), colA4, rowA3)
    ewB = _mlp(edge_attr[_EA:], Wm1, bm1r, Wm2, bm2r, _EB, 40)
    scB = _sc_scatter_b(xif, ewB.reshape(2 * _EB, _FH), colB4, rowB3)
    return _lin2(scA.reshape(2, _N, _FH), scB.reshape(2, _N, _FH),
                 W2, b2.reshape(1, _H))


# speedup vs baseline: 1.0675x; 1.0092x over previous
import functools

import jax
import jax.numpy as jnp
from jax import lax
from jax.experimental import pallas as pl
from jax.experimental.pallas import tpu as pltpu
from jax.experimental.pallas import tpu_sc as plsc

_N = 10000
_E = 160000
_EA = 24320
_EB = _E - _EA
_H = 256
_G = 16
_F = 256
_FH = 128
_NS = 16
_K = 40
_ZCH = _N // _K


def _pre_body(x_ref, ea_ref, w1_ref, b1_ref, wm1_ref, bm1_ref, wm2_ref,
              bm2_ref, xi_ref, ew_ref):
    r = lax.dot_general(x_ref[...], w1_ref[...], (((1,), (1,)), ((), ())),
                        preferred_element_type=jnp.float32)
    r = r + b1_ref[...]
    xi_ref[0] = r[:, :_FH]
    xi_ref[1] = r[:, _FH:]
    h = lax.dot_general(ea_ref[...], wm1_ref[...], (((1,), (1,)), ((), ())),
                        preferred_element_type=jnp.float32)
    h = jnp.maximum(h + bm1_ref[...], 0.0)
    ew = lax.dot_general(h, wm2_ref[...], (((1,), (1,)), ((), ())),
                         preferred_element_type=jnp.float32)
    ew = jnp.maximum(ew + bm2_ref[...], 0.0)
    ew_ref[0] = ew[:, :_FH]
    ew_ref[1] = ew[:, _FH:]


def _pre(x, eaA, W1, b1r, Wm1, bm1r, Wm2, bm2r):
    nb = 5
    return pl.pallas_call(
        _pre_body,
        grid=(nb,),
        in_specs=[
            pl.BlockSpec((_N // nb, _H), lambda i: (i, 0)),
            pl.BlockSpec((_EA // nb, _G), lambda i: (i, 0)),
            pl.BlockSpec((_F, _H), lambda i: (0, 0)),
            pl.BlockSpec((1, _F), lambda i: (0, 0)),
            pl.BlockSpec((_F, _G), lambda i: (0, 0)),
            pl.BlockSpec((1, _F), lambda i: (0, 0)),
            pl.BlockSpec((_F, _F), lambda i: (0, 0)),
            pl.BlockSpec((1, _F), lambda i: (0, 0)),
        ],
        out_specs=[
            pl.BlockSpec((2, _N // nb, _FH), lambda i: (0, i, 0)),
            pl.BlockSpec((2, _EA // nb, _FH), lambda i: (0, i, 0)),
        ],
        out_shape=[
            jax.ShapeDtypeStruct((2, _N, _FH), jnp.float32),
            jax.ShapeDtypeStruct((2, _EA, _FH), jnp.float32),
        ],
    )(x, eaA, W1, b1r, Wm1, bm1r, Wm2, bm2r)


def _mlp_body(ea_ref, wm1_ref, bm1_ref, wm2_ref, bm2_ref, out_ref):
    h = lax.dot_general(ea_ref[...], wm1_ref[...], (((1,), (1,)), ((), ())),
                        preferred_element_type=jnp.float32)
    h = jnp.maximum(h + bm1_ref[...], 0.0)
    ew = lax.dot_general(h, wm2_ref[...], (((1,), (1,)), ((), ())),
                         preferred_element_type=jnp.float32)
    ew = jnp.maximum(ew + bm2_ref[...], 0.0)
    out_ref[0] = ew[:, :_FH]
    out_ref[1] = ew[:, _FH:]


def _mlp(edge_attr, Wm1, bm1r, Wm2, bm2r, e, eb):
    return pl.pallas_call(
        _mlp_body,
        grid=(eb,),
        in_specs=[
            pl.BlockSpec((e // eb, _G), lambda i: (i, 0)),
            pl.BlockSpec((_F, _G), lambda i: (0, 0)),
            pl.BlockSpec((1, _F), lambda i: (0, 0)),
            pl.BlockSpec((_F, _F), lambda i: (0, 0)),
            pl.BlockSpec((1, _F), lambda i: (0, 0)),
        ],
        out_specs=pl.BlockSpec((2, e // eb, _FH), lambda i: (0, i, 0)),
        out_shape=jax.ShapeDtypeStruct((2, e, _FH), jnp.float32),
    )(edge_attr, Wm1, bm1r, Wm2, bm2r)


def _sc_body(eh, nch, xi_hbm, ew_hbm, col_hbm, row_hbm, out_hbm,
             colr, rowr, xiv, ewv, accum,
             isem0, isem1, isem2, isem3, gsem0, gsem1, esem0, esem1,
             ssem0, ssem1):
    c = lax.axis_index("c")
    s = lax.axis_index("s")
    isems = (isem0, isem1, isem2, isem3)
    gsems = (gsem0, gsem1)
    esems = (esem0, esem1)
    ssems = (ssem0, ssem1)

    ept = eh // _NS
    ebase = c * eh + s * ept

    def idx_start(j, q):
        pltpu.async_copy(col_hbm.at[c, s, j], colr.at[q], isems[q])
        pltpu.async_copy(row_hbm.at[s, j], rowr.at[q], isems[q])

    def idx_wait(j, q):
        pltpu.make_async_copy(col_hbm.at[c, s, j], colr.at[q],
                              isems[q]).wait()
        pltpu.make_async_copy(row_hbm.at[s, j], rowr.at[q],
                              isems[q]).wait()

    def data_start(j, b, q):
        pltpu.async_copy(xi_hbm.at[colr.at[q]], xiv.at[b], gsems[b])
        pltpu.async_copy(ew_hbm.at[pl.ds(ebase + j * _K, _K)], ewv.at[b],
                         esems[b])

    def scat_wait(b, q):
        pltpu.make_async_copy(xiv.at[b], accum.at[rowr.at[q]],
                              ssems[b]).wait()

    def data_finish(j, b, q):
        pltpu.make_async_copy(xi_hbm.at[colr.at[q]], xiv.at[b],
                              gsems[b]).wait()
        pltpu.make_async_copy(ew_hbm.at[pl.ds(ebase + j * _K, _K)],
                              ewv.at[b], esems[b]).wait()
        xb = xiv.at[b]
        eb = ewv.at[b]

        @plsc.parallel_loop(0, _K, unroll=4)
        def _(r):
            for g in range(8):
                sl = pl.ds(g * 16, 16)
                xb[r, sl] = xb[r, sl] * eb[r, sl]

        pltpu.async_copy(xb, accum.at[rowr.at[q]], ssems[b], add=True)

    idx_start(0, 0)
    idx_start(1, 1)
    idx_start(2, 2)

    @plsc.parallel_loop(0, _K, unroll=4)
    def _(r):
        for g in range(8):
            ewv[0, r, pl.ds(g * 16, 16)] = jnp.zeros((16,), jnp.float32)

    def zchunk(j, carry):
        z = s + _NS * j

        @pl.when(z < _ZCH)
        def _():
            pltpu.async_copy(ewv.at[0], accum.at[pl.ds(z * _K, _K)], gsem0)

        return carry

    nz = (_ZCH + _NS - 1) // _NS
    lax.fori_loop(0, nz, zchunk, 0)

    def zdrain(j, carry):
        z = s + _NS * j

        @pl.when(z < _ZCH)
        def _():
            pltpu.make_async_copy(ewv.at[0], accum.at[pl.ds(z * _K, _K)],
                                  gsem0).wait()

        return carry

    lax.fori_loop(0, nz, zdrain, 0)
    plsc.subcore_barrier()

    idx_wait(0, 0)
    data_start(0, 0, 0)

    def step(t, carry):
        j0 = 4 * t
        for u in range(4):
            j = j0 + u

            @pl.when(j >= 1)
            def _():
                scat_wait((u + 1) % 2, (u + 3) % 4)

            @pl.when(j + 1 < nch)
            def _():
                idx_wait(j + 1, (u + 1) % 4)
                data_start(j + 1, (u + 1) % 2, (u + 1) % 4)

            data_finish(j, u % 2, u)

            @pl.when(j + 3 < nch)
            def _():
                idx_start(j + 3, (u + 3) % 4)

        return carry

    lax.fori_loop(0, nch // 4, step, 0)
    if nch % 4 == 2:
        j = nch - 2
        scat_wait(1, 3)
        idx_wait(j + 1, 1)
        data_start(j + 1, 1, 1)
        data_finish(j, 0, 0)
        scat_wait(0, 0)
        data_finish(j + 1, 1, 1)
        scat_wait(1, 1)
    else:
        scat_wait(1, 3)
    plsc.subcore_barrier()

    r0 = s * 624

    @pl.when(s < _NS - 1)
    def _():
        pltpu.sync_copy(accum.at[pl.ds(r0, 624)],
                        out_hbm.at[pl.ds(c * _N + r0, 624)])

    @pl.when(s == _NS - 1)
    def _():
        pltpu.sync_copy(accum.at[pl.ds(15 * 624, _N - 15 * 624)],
                        out_hbm.at[pl.ds(c * _N + 15 * 624, _N - 15 * 624)])


def _make_sc_scatter(eh):
    nch = eh // _NS // _K
    return functools.partial(
        pl.kernel,
        mesh=plsc.VectorSubcoreMesh(core_axis_name="c",
                                    subcore_axis_name="s"),
        out_type=jax.ShapeDtypeStruct((2 * _N, _FH), jnp.float32),
        scratch_types=[
            pltpu.VMEM((4, _K), jnp.int32),
            pltpu.VMEM((4, _K), jnp.int32),
            pltpu.VMEM((2, _K, _FH), jnp.float32),
            pltpu.VMEM((2, _K, _FH), jnp.float32),
            pltpu.VMEM_SHARED((_N, _FH), jnp.float32),
        ] + [pltpu.SemaphoreType.DMA] * 10,
    )(functools.partial(_sc_body, eh, nch))


_sc_scatter_a = _make_sc_scatter(_EA)
_sc_scatter_b = _make_sc_scatter(_EB)


def _lin2_body(sa_ref, sb_ref, w2_ref, b2_ref, out_ref):
    scat = jnp.concatenate([sa_ref[0] + sb_ref[0], sa_ref[1] + sb_ref[1]],
                           axis=1)
    r = lax.dot_general(scat, w2_ref[...], (((1,), (1,)), ((), ())),
                        preferred_element_type=jnp.float32)
    out_ref[...] = r + b2_ref[...]


def _lin2(sa3, sb3, W2, b2r):
    nb = 5
    return pl.pallas_call(
        _lin2_body,
        grid=(nb,),
        in_specs=[
            pl.BlockSpec((2, _N // nb, _FH), lambda i: (0, i, 0)),
            pl.BlockSpec((2, _N // nb, _FH), lambda i: (0, i, 0)),
            pl.BlockSpec((_H, _F), lambda i: (0, 0)),
            pl.BlockSpec((1, _H), lambda i: (0, 0)),
        ],
        out_specs=pl.BlockSpec((_N // nb, _H), lambda i: (i, 0)),
        out_shape=jax.ShapeDtypeStruct((_N, _H), jnp.float32),
    )(sa3, sb3, W2, b2r)


def kernel(x, edge_index, edge_attr, W1, b1, W2, b2, Wm1, bm1, Wm2, bm2):
    row = edge_index[0]
    col = edge_index[1]
    colA = jnp.stack([col[:_EA], col[:_EA] + _N])
    colB = jnp.stack([col[_EA:], col[_EA:] + _N])
    colA4 = colA.reshape(2, _NS, -1, _K)
    colB4 = colB.reshape(2, _NS, -1, _K)
    rowA3 = row[:_EA].reshape(_NS, -1, _K)
    rowB3 = row[_EA:].reshape(_NS, -1, _K)

    bm1r = bm1.reshape(1, _F)
    bm2r = bm2.reshape(1, _F)

    xi, ewA = _pre(x, edge_attr[:_EA], W1, b1.reshape(1, _F),
                   Wm1, bm1r, Wm2, bm2r)
    xif = xi.reshape(2 * _N, _FH)
    scA = _sc_scatter_a(xif, ewA.reshape(2 * _EA, _FH), colA4, rowA3)
    ewB = _mlp(edge_attr[_EA:], Wm1, bm1r, Wm2, bm2r, _EB, 40)
    scB = _sc_scatter_b(xif, ewB.reshape(2 * _EB, _FH), colB4, rowB3)
    return _lin2(scA.reshape(2, _N, _FH), scB.reshape(2, _N, _FH),
                 W2, b2.reshape(1, _H))
